# baseline (speedup 1.0000x reference)
import functools

import jax
import jax.numpy as jnp
from jax import lax
from jax.experimental import pallas as pl
from jax.experimental.pallas import tpu as pltpu
from jax.experimental.pallas import tpu_sc as plsc

N_SRC0 = 10000
N_DST0 = 4000
N_DST1 = 2000
E0 = 128000
E1 = 64000
D = 128
D2 = 256

_L = 16
_CHUNK = 128
_ROWCH = 200


def _seg_max_sc(m_flat, src, dst, *, n_src, n_dst, n_fchunk, n_group):
  E = src.shape[0]
  fpc = n_fchunk // 2
  assert 16 % fpc == 0
  n_g = 16 // fpc
  assert n_g == n_group
  epg = E // n_group
  assert epg % _CHUNK == 0
  n_chunks = epg // _CHUNK
  rows_pg = n_dst // n_group
  assert rows_pg % _ROWCH == 0

  mesh = plsc.VectorSubcoreMesh(core_axis_name="c", subcore_axis_name="s",
                                num_cores=2, num_subcores=16)

  def body(m_h, src_h, dst_h, out_h, part_h,
           acc, srcv, dstv, idxv, rows, cmb, obuf, sem):
    c = lax.axis_index("c")
    s = lax.axis_index("s")
    fl = lax.rem(s, fpc)
    f = c * fpc + fl
    g = lax.div(s, fpc)

    col16 = lax.iota(jnp.int32, _L)
    neg_inf = jnp.full((_L,), -jnp.inf, jnp.float32)

    def init_body(r, _):
      acc[r, :] = neg_inf
      return 0
    lax.fori_loop(0, n_dst, init_body, 0)

    def idx_body(j, _):
      s16 = srcv[pl.ds(j * _L, _L)]
      idxv[pl.ds(j * _L, _L)] = s16 * n_fchunk + f
      return 0

    def edge_body(i, _):
      bi = jnp.zeros((_L,), jnp.int32) + i
      dvec = plsc.load_gather(dstv, [bi])
      cur = plsc.load_gather(acc, [dvec, col16])
      msg = rows[i, :]
      plsc.store_scatter(acc, [dvec, col16], jnp.maximum(cur, msg))
      return 0

    def chunk_body(ci, _):
      eb = g * epg + ci * _CHUNK
      pltpu.sync_copy(src_h.at[pl.ds(eb, _CHUNK)], srcv)
      pltpu.sync_copy(dst_h.at[pl.ds(eb, _CHUNK)], dstv)
      lax.fori_loop(0, _CHUNK // _L, idx_body, 0)
      pltpu.async_copy(m_h.at[idxv], rows, sem).wait()
      lax.fori_loop(0, _CHUNK, edge_body, 0)
      return 0
    lax.fori_loop(0, n_chunks, chunk_body, 0)

    pltpu.sync_copy(acc, part_h.at[g, f])
    plsc.subcore_barrier()

    r0 = g * rows_pg

    def q_body(q, _):
      rq = r0 + q * _ROWCH
      for gg in range(n_group):
        pltpu.sync_copy(part_h.at[gg, f, pl.ds(rq, _ROWCH)], cmb.at[gg])

      def row_body(r, _):
        v = cmb[0, r, :]
        for gg in range(1, n_group):
          v = jnp.maximum(v, cmb[gg, r, :])
        obuf[r, :] = jnp.where(v == -jnp.inf, 0.0, v)
        return 0
      lax.fori_loop(0, _ROWCH, row_body, 0)
      pltpu.sync_copy(obuf, out_h.at[f, pl.ds(rq, _ROWCH)])
      return 0
    lax.fori_loop(0, rows_pg // _ROWCH, q_body, 0)

  scratch = [
      pltpu.VMEM((n_dst, _L), jnp.float32),
      pltpu.VMEM((_CHUNK,), jnp.int32),
      pltpu.VMEM((_CHUNK,), jnp.int32),
      pltpu.VMEM((_CHUNK,), jnp.int32),
      pltpu.VMEM((_CHUNK, _L), jnp.float32),
      pltpu.VMEM((n_group, _ROWCH, _L), jnp.float32),
      pltpu.VMEM((_ROWCH, _L), jnp.float32),
      pltpu.SemaphoreType.DMA,
  ]
  out, _ = pl.kernel(
      body,
      out_type=(jax.ShapeDtypeStruct((n_fchunk, n_dst, _L), jnp.float32),
                jax.ShapeDtypeStruct((n_group, n_fchunk, n_dst, _L),
                                     jnp.float32)),
      mesh=mesh,
      scratch_types=scratch,
      compiler_params=pltpu.CompilerParams(
          needs_layout_passes=False, use_tc_tiling_on_sc=False),
  )(m_flat, src, dst)
  return out


def _tc_pool1(h, wt, b2d):
  def body(h_ref, w_ref, b_ref, o_ref):
    o_ref[...] = jnp.maximum(
        jnp.dot(h_ref[...], w_ref[...], preferred_element_type=jnp.float32)
        + b_ref[...], 0.0)
  return pl.pallas_call(
      body, out_shape=jax.ShapeDtypeStruct(h.shape, jnp.float32))(h, wt, b2d)


def _tc_mid(h4, nfm, ws_t, wn_fm, b1_2d, wp2_t, bp2_2d):
  def body(h_ref, n_ref, ws_ref, wn_ref, b_ref, wp_ref, bp_ref,
           h1_ref, m2_ref):
    t = jnp.dot(h_ref[...], ws_ref[...],
                preferred_element_type=jnp.float32) + b_ref[...]
    for fc in range(n_ref.shape[0]):
      t = t + jnp.dot(n_ref[fc], wn_ref[fc],
                      preferred_element_type=jnp.float32)
    h1 = jnp.maximum(t, 0.0)
    h1_ref[...] = h1
    m2_ref[...] = jnp.maximum(
        jnp.dot(h1, wp_ref[...], preferred_element_type=jnp.float32)
        + bp_ref[...], 0.0)
  return pl.pallas_call(
      body,
      out_shape=(jax.ShapeDtypeStruct((N_DST0, D2), jnp.float32),
                 jax.ShapeDtypeStruct((N_DST0, D2), jnp.float32)),
  )(h4, nfm, ws_t, wn_fm, b1_2d, wp2_t, bp2_2d)


def _tc_out(hdst, h1s, nfm, ws_t, wn_fm, b2_2d):
  def body(hd_ref, h_ref, n_ref, ws_ref, wn_ref, b_ref, o_ref):
    t = (jnp.dot(h_ref[...], ws_ref[...],
                 preferred_element_type=jnp.float32)
         + b_ref[...] + hd_ref[...])
    for fc in range(n_ref.shape[0]):
      t = t + jnp.dot(n_ref[fc], wn_ref[fc],
                      preferred_element_type=jnp.float32)
    o_ref[...] = t
  return pl.pallas_call(
      body, out_shape=jax.ShapeDtypeStruct((N_DST1, D), jnp.float32),
  )(hdst, h1s, nfm, ws_t, wn_fm, b2_2d)


def kernel(h_item, h_item_dst, src0, dst0, src1, dst1,
           Wp1, bp1, Wn1, Ws1, b1, Wp2, bp2, Wn2, Ws2, b2):
  m1 = _tc_pool1(h_item, Wp1.T, bp1.reshape(1, D))
  neigh1_fm = _seg_max_sc(m1.reshape(N_SRC0 * (D // _L), _L), src0, dst0,
                          n_src=N_SRC0, n_dst=N_DST0,
                          n_fchunk=D // _L, n_group=4)
  h1, m2 = _tc_mid(h_item[:N_DST0], neigh1_fm, Ws1.T,
                   Wn1.T.reshape(D // _L, _L, D2), b1.reshape(1, D2),
                   Wp2.T, bp2.reshape(1, D2))
  neigh2_fm = _seg_max_sc(m2.reshape(N_DST0 * (D2 // _L), _L), src1, dst1,
                          n_src=N_DST0, n_dst=N_DST1,
                          n_fchunk=D2 // _L, n_group=2)
  return _tc_out(h_item_dst, h1[:N_DST1], neigh2_fm, Ws2.T,
                 Wn2.T.reshape(D2 // _L, _L, D), b2.reshape(1, D))


# speedup vs baseline: 2.8533x; 2.8533x over previous
import functools

import jax
import jax.numpy as jnp
from jax import lax
from jax.experimental import pallas as pl
from jax.experimental.pallas import tpu as pltpu
from jax.experimental.pallas import tpu_sc as plsc

N_SRC0 = 10000
N_DST0 = 4000
N_DST1 = 2000
E0 = 128000
E1 = 64000
D = 128
D2 = 256

_L = 16
_CHUNK = 256
_ROWCH = 200


def _seg_max_sc(m_flat, src, dst, *, n_src, n_dst, n_fchunk, n_group):
  E = src.shape[0]
  fpc = n_fchunk // 2
  assert 16 % fpc == 0
  n_g = 16 // fpc
  assert n_g == n_group
  epg = E // n_group
  assert epg % _CHUNK == 0
  n_chunks = epg // _CHUNK
  rows_pg = n_dst // n_group
  assert rows_pg % _ROWCH == 0

  mesh = plsc.VectorSubcoreMesh(core_axis_name="c", subcore_axis_name="s",
                                num_cores=2, num_subcores=16)

  def body(m_h, src_h, dst_h, out_h, part_h,
           acc, srcv, dstv, idxv, rows, cmb, obuf, semg, sese, sesd):
    c = lax.axis_index("c")
    s = lax.axis_index("s")
    fl = lax.rem(s, fpc)
    f = c * fpc + fl
    g = lax.div(s, fpc)

    col16 = lax.iota(jnp.int32, _L)
    neg_inf = jnp.full((_L,), -jnp.inf, jnp.float32)

    def init_body(r, _):
      acc[r, :] = neg_inf
      return 0
    lax.fori_loop(0, n_dst, init_body, 0)

    def start_edges(ci, p):
      eb = g * epg + ci * _CHUNK
      pltpu.async_copy(src_h.at[pl.ds(eb, _CHUNK)], srcv.at[p], sese.at[p])
      pltpu.async_copy(dst_h.at[pl.ds(eb, _CHUNK)], dstv.at[p], sesd.at[p])

    def start_gather(p):
      def idx_body(j, _):
        s16 = srcv[p, pl.ds(j * _L, _L)]
        idxv[p, pl.ds(j * _L, _L)] = s16 * n_fchunk + f
        return 0
      lax.fori_loop(0, _CHUNK // _L, idx_body, 0)
      pltpu.async_copy(m_h.at[idxv.at[p]], rows.at[p], semg.at[p])

    def wait_edges(p):
      pltpu.make_async_copy(src_h.at[pl.ds(0, _CHUNK)], srcv.at[p],
                            sese.at[p]).wait()
      pltpu.make_async_copy(dst_h.at[pl.ds(0, _CHUNK)], dstv.at[p],
                            sesd.at[p]).wait()

    def accumulate(p):
      pltpu.make_async_copy(m_h.at[idxv.at[p]], rows.at[p], semg.at[p]).wait()

      def group_body(i, _):
        d16 = dstv[p, pl.ds(i * _L, _L)]
        for l in range(_L):
          dvec = jnp.take(d16, jnp.full((_L,), l, jnp.int32))
          cur = plsc.load_gather(acc, [dvec, col16])
          msg = rows[p, i * _L + l, :]
          plsc.store_scatter(acc, [dvec, col16], jnp.maximum(cur, msg))
        return 0
      lax.fori_loop(0, _CHUNK // _L, group_body, 0)

    start_edges(0, 0)
    start_edges(1, 1)
    wait_edges(0)
    start_gather(0)

    def chunk_body(ci, _):
      p = lax.rem(ci, 2)
      pn = lax.rem(ci + 1, 2)

      @pl.when(ci + 1 < n_chunks)
      def _():
        wait_edges(pn)
        start_gather(pn)

      accumulate(p)

      @pl.when(ci + 2 < n_chunks)
      def _():
        start_edges(ci + 2, p)
      return 0
    lax.fori_loop(0, n_chunks, chunk_body, 0)

    pltpu.sync_copy(acc, part_h.at[g, f])
    plsc.subcore_barrier()

    r0 = g * rows_pg

    def q_body(q, _):
      rq = r0 + q * _ROWCH
      for gg in range(n_group):
        pltpu.sync_copy(part_h.at[gg, f, pl.ds(rq, _ROWCH)], cmb.at[gg])

      def row_body(r, _):
        v = cmb[0, r, :]
        for gg in range(1, n_group):
          v = jnp.maximum(v, cmb[gg, r, :])
        obuf[r, :] = jnp.where(v == -jnp.inf, 0.0, v)
        return 0
      lax.fori_loop(0, _ROWCH, row_body, 0)
      pltpu.sync_copy(obuf, out_h.at[f, pl.ds(rq, _ROWCH)])
      return 0
    lax.fori_loop(0, rows_pg // _ROWCH, q_body, 0)

  scratch = [
      pltpu.VMEM((n_dst, _L), jnp.float32),
      pltpu.VMEM((2, _CHUNK), jnp.int32),
      pltpu.VMEM((2, _CHUNK), jnp.int32),
      pltpu.VMEM((2, _CHUNK), jnp.int32),
      pltpu.VMEM((2, _CHUNK, _L), jnp.float32),
      pltpu.VMEM((n_group, _ROWCH, _L), jnp.float32),
      pltpu.VMEM((_ROWCH, _L), jnp.float32),
      pltpu.SemaphoreType.DMA((2,)),
      pltpu.SemaphoreType.DMA((2,)),
      pltpu.SemaphoreType.DMA((2,)),
  ]
  out, _ = pl.kernel(
      body,
      out_type=(jax.ShapeDtypeStruct((n_fchunk, n_dst, _L), jnp.float32),
                jax.ShapeDtypeStruct((n_group, n_fchunk, n_dst, _L),
                                     jnp.float32)),
      mesh=mesh,
      scratch_types=scratch,
      compiler_params=pltpu.CompilerParams(
          needs_layout_passes=False, use_tc_tiling_on_sc=False),
  )(m_flat, src, dst)
  return out


def _tc_pool1(h, wt, b2d):
  def body(h_ref, w_ref, b_ref, o_ref):
    o_ref[...] = jnp.maximum(
        jnp.dot(h_ref[...], w_ref[...], preferred_element_type=jnp.float32)
        + b_ref[...], 0.0)
  return pl.pallas_call(
      body, out_shape=jax.ShapeDtypeStruct(h.shape, jnp.float32))(h, wt, b2d)


def _tc_mid(h4, nfm, ws_t, wn_fm, b1_2d, wp2_t, bp2_2d):
  def body(h_ref, n_ref, ws_ref, wn_ref, b_ref, wp_ref, bp_ref,
           h1_ref, m2_ref):
    t = jnp.dot(h_ref[...], ws_ref[...],
                preferred_element_type=jnp.float32) + b_ref[...]
    for fc in range(n_ref.shape[0]):
      t = t + jnp.dot(n_ref[fc], wn_ref[fc],
                      preferred_element_type=jnp.float32)
    h1 = jnp.maximum(t, 0.0)
    h1_ref[...] = h1
    m2_ref[...] = jnp.maximum(
        jnp.dot(h1, wp_ref[...], preferred_element_type=jnp.float32)
        + bp_ref[...], 0.0)
  return pl.pallas_call(
      body,
      out_shape=(jax.ShapeDtypeStruct((N_DST0, D2), jnp.float32),
                 jax.ShapeDtypeStruct((N_DST0, D2), jnp.float32)),
  )(h4, nfm, ws_t, wn_fm, b1_2d, wp2_t, bp2_2d)


def _tc_out(hdst, h1s, nfm, ws_t, wn_fm, b2_2d):
  def body(hd_ref, h_ref, n_ref, ws_ref, wn_ref, b_ref, o_ref):
    t = (jnp.dot(h_ref[...], ws_ref[...],
                 preferred_element_type=jnp.float32)
         + b_ref[...] + hd_ref[...])
    for fc in range(n_ref.shape[0]):
      t = t + jnp.dot(n_ref[fc], wn_ref[fc],
                      preferred_element_type=jnp.float32)
    o_ref[...] = t
  return pl.pallas_call(
      body, out_shape=jax.ShapeDtypeStruct((N_DST1, D), jnp.float32),
  )(hdst, h1s, nfm, ws_t, wn_fm, b2_2d)


def kernel(h_item, h_item_dst, src0, dst0, src1, dst1,
           Wp1, bp1, Wn1, Ws1, b1, Wp2, bp2, Wn2, Ws2, b2):
  m1 = _tc_pool1(h_item, Wp1.T, bp1.reshape(1, D))
  neigh1_fm = _seg_max_sc(m1.reshape(N_SRC0 * (D // _L), _L), src0, dst0,
                          n_src=N_SRC0, n_dst=N_DST0,
                          n_fchunk=D // _L, n_group=4)
  h1, m2 = _tc_mid(h_item[:N_DST0], neigh1_fm, Ws1.T,
                   Wn1.T.reshape(D // _L, _L, D2), b1.reshape(1, D2),
                   Wp2.T, bp2.reshape(1, D2))
  neigh2_fm = _seg_max_sc(m2.reshape(N_DST0 * (D2 // _L), _L), src1, dst1,
                          n_src=N_DST0, n_dst=N_DST1,
                          n_fchunk=D2 // _L, n_group=2)
  return _tc_out(h_item_dst, h1[:N_DST1], neigh2_fm, Ws2.T,
                 Wn2.T.reshape(D2 // _L, _L, D), b2.reshape(1, D))


# speedup vs baseline: 2.8976x; 1.0155x over previous
import functools

import jax
import jax.numpy as jnp
from jax import lax
from jax.experimental import pallas as pl
from jax.experimental.pallas import tpu as pltpu
from jax.experimental.pallas import tpu_sc as plsc

N_SRC0 = 10000
N_DST0 = 4000
N_DST1 = 2000
E0 = 128000
E1 = 64000
D = 128
D2 = 256

_L = 16
_CHUNK = 256
_ROWCH = 200


def _seg_max_sc(m_flat, src, dst, *, n_src, n_dst, n_fchunk, n_group):
  E = src.shape[0]
  fpc = n_fchunk // 2
  assert 16 % fpc == 0
  n_g = 16 // fpc
  assert n_g == n_group
  epg = E // n_group
  assert epg % _CHUNK == 0
  n_chunks = epg // _CHUNK
  rows_pg = n_dst // n_group
  assert rows_pg % _ROWCH == 0

  mesh = plsc.VectorSubcoreMesh(core_axis_name="c", subcore_axis_name="s",
                                num_cores=2, num_subcores=16)

  def body(m_h, src_h, dst_h, out_h, part_h,
           acc, srcv, dstv, idxv, rows, cmb, obuf, semg, sese, sesd):
    c = lax.axis_index("c")
    s = lax.axis_index("s")
    fl = lax.rem(s, fpc)
    f = c * fpc + fl
    g = lax.div(s, fpc)

    col16 = lax.iota(jnp.int32, _L)
    neg_inf = jnp.full((_L,), -jnp.inf, jnp.float32)

    def init_body(r, _):
      acc[r, :] = neg_inf
      return 0
    lax.fori_loop(0, n_dst, init_body, 0)

    def start_edges(ci, p):
      eb = g * epg + ci * _CHUNK
      pltpu.async_copy(src_h.at[pl.ds(eb, _CHUNK)], srcv.at[p], sese.at[p])
      pltpu.async_copy(dst_h.at[pl.ds(eb, _CHUNK)], dstv.at[p], sesd.at[p])

    def start_gather(p):
      def idx_body(j, _):
        s16 = srcv[p, pl.ds(j * _L, _L)]
        idxv[p, pl.ds(j * _L, _L)] = s16 * n_fchunk + f
        return 0
      lax.fori_loop(0, _CHUNK // _L, idx_body, 0, unroll=4)
      pltpu.async_copy(m_h.at[idxv.at[p]], rows.at[p], semg.at[p])

    def wait_edges(p):
      pltpu.make_async_copy(src_h.at[pl.ds(0, _CHUNK)], srcv.at[p],
                            sese.at[p]).wait()
      pltpu.make_async_copy(dst_h.at[pl.ds(0, _CHUNK)], dstv.at[p],
                            sesd.at[p]).wait()

    def accumulate(p):
      pltpu.make_async_copy(m_h.at[idxv.at[p]], rows.at[p], semg.at[p]).wait()

      def group_body(i, _):
        d16 = dstv[p, pl.ds(i * _L, _L)]
        for l in range(_L):
          dvec = jnp.take(d16, jnp.full((_L,), l, jnp.int32))
          cur = plsc.load_gather(acc, [dvec, col16])
          msg = rows[p, i * _L + l, :]
          plsc.store_scatter(acc, [dvec, col16], jnp.maximum(cur, msg))
        return 0
      lax.fori_loop(0, _CHUNK // _L, group_body, 0, unroll=4)

    start_edges(0, 0)
    start_edges(1, 1)
    wait_edges(0)
    start_gather(0)

    def chunk_body(ci, _):
      p = lax.rem(ci, 2)
      pn = lax.rem(ci + 1, 2)

      @pl.when(ci + 1 < n_chunks)
      def _():
        wait_edges(pn)
        start_gather(pn)

      accumulate(p)

      @pl.when(ci + 2 < n_chunks)
      def _():
        start_edges(ci + 2, p)
      return 0
    lax.fori_loop(0, n_chunks, chunk_body, 0)

    pltpu.sync_copy(acc, part_h.at[g, f])
    plsc.subcore_barrier()

    r0 = g * rows_pg

    def q_body(q, _):
      rq = r0 + q * _ROWCH
      for gg in range(n_group):
        pltpu.sync_copy(part_h.at[gg, f, pl.ds(rq, _ROWCH)], cmb.at[gg])

      def row_body(r, _):
        v = cmb[0, r, :]
        for gg in range(1, n_group):
          v = jnp.maximum(v, cmb[gg, r, :])
        obuf[r, :] = jnp.where(v == -jnp.inf, 0.0, v)
        return 0
      lax.fori_loop(0, _ROWCH, row_body, 0)
      pltpu.sync_copy(obuf, out_h.at[f, pl.ds(rq, _ROWCH)])
      return 0
    lax.fori_loop(0, rows_pg // _ROWCH, q_body, 0)

  scratch = [
      pltpu.VMEM((n_dst, _L), jnp.float32),
      pltpu.VMEM((2, _CHUNK), jnp.int32),
      pltpu.VMEM((2, _CHUNK), jnp.int32),
      pltpu.VMEM((2, _CHUNK), jnp.int32),
      pltpu.VMEM((2, _CHUNK, _L), jnp.float32),
      pltpu.VMEM((n_group, _ROWCH, _L), jnp.float32),
      pltpu.VMEM((_ROWCH, _L), jnp.float32),
      pltpu.SemaphoreType.DMA((2,)),
      pltpu.SemaphoreType.DMA((2,)),
      pltpu.SemaphoreType.DMA((2,)),
  ]
  out, _ = pl.kernel(
      body,
      out_type=(jax.ShapeDtypeStruct((n_fchunk, n_dst, _L), jnp.float32),
                jax.ShapeDtypeStruct((n_group, n_fchunk, n_dst, _L),
                                     jnp.float32)),
      mesh=mesh,
      scratch_types=scratch,
      compiler_params=pltpu.CompilerParams(
          needs_layout_passes=False, use_tc_tiling_on_sc=False),
  )(m_flat, src, dst)
  return out


def _tc_pool1(h, wt, b2d):
  def body(h_ref, w_ref, b_ref, o_ref):
    o_ref[...] = jnp.maximum(
        jnp.dot(h_ref[...], w_ref[...], preferred_element_type=jnp.float32)
        + b_ref[...], 0.0)
  return pl.pallas_call(
      body, out_shape=jax.ShapeDtypeStruct(h.shape, jnp.float32))(h, wt, b2d)


def _tc_mid(h4, nfm, ws_t, wn_fm, b1_2d, wp2_t, bp2_2d):
  def body(h_ref, n_ref, ws_ref, wn_ref, b_ref, wp_ref, bp_ref,
           h1_ref, m2_ref):
    t = jnp.dot(h_ref[...], ws_ref[...],
                preferred_element_type=jnp.float32) + b_ref[...]
    for fc in range(n_ref.shape[0]):
      t = t + jnp.dot(n_ref[fc], wn_ref[fc],
                      preferred_element_type=jnp.float32)
    h1 = jnp.maximum(t, 0.0)
    h1_ref[...] = h1
    m2_ref[...] = jnp.maximum(
        jnp.dot(h1, wp_ref[...], preferred_element_type=jnp.float32)
        + bp_ref[...], 0.0)
  return pl.pallas_call(
      body,
      out_shape=(jax.ShapeDtypeStruct((N_DST0, D2), jnp.float32),
                 jax.ShapeDtypeStruct((N_DST0, D2), jnp.float32)),
  )(h4, nfm, ws_t, wn_fm, b1_2d, wp2_t, bp2_2d)


def _tc_out(hdst, h1s, nfm, ws_t, wn_fm, b2_2d):
  def body(hd_ref, h_ref, n_ref, ws_ref, wn_ref, b_ref, o_ref):
    t = (jnp.dot(h_ref[...], ws_ref[...],
                 preferred_element_type=jnp.float32)
         + b_ref[...] + hd_ref[...])
    for fc in range(n_ref.shape[0]):
      t = t + jnp.dot(n_ref[fc], wn_ref[fc],
                      preferred_element_type=jnp.float32)
    o_ref[...] = t
  return pl.pallas_call(
      body, out_shape=jax.ShapeDtypeStruct((N_DST1, D), jnp.float32),
  )(hdst, h1s, nfm, ws_t, wn_fm, b2_2d)


def kernel(h_item, h_item_dst, src0, dst0, src1, dst1,
           Wp1, bp1, Wn1, Ws1, b1, Wp2, bp2, Wn2, Ws2, b2):
  m1 = _tc_pool1(h_item, Wp1.T, bp1.reshape(1, D))
  neigh1_fm = _seg_max_sc(m1.reshape(N_SRC0 * (D // _L), _L), src0, dst0,
                          n_src=N_SRC0, n_dst=N_DST0,
                          n_fchunk=D // _L, n_group=4)
  h1, m2 = _tc_mid(h_item[:N_DST0], neigh1_fm, Ws1.T,
                   Wn1.T.reshape(D // _L, _L, D2), b1.reshape(1, D2),
                   Wp2.T, bp2.reshape(1, D2))
  neigh2_fm = _seg_max_sc(m2.reshape(N_DST0 * (D2 // _L), _L), src1, dst1,
                          n_src=N_DST0, n_dst=N_DST1,
                          n_fchunk=D2 // _L, n_group=2)
  return _tc_out(h_item_dst, h1[:N_DST1], neigh2_fm, Ws2.T,
                 Wn2.T.reshape(D2 // _L, _L, D), b2.reshape(1, D))


# speedup vs baseline: 3.1426x; 1.0846x over previous
import functools

import jax
import jax.numpy as jnp
from jax import lax
from jax.experimental import pallas as pl
from jax.experimental.pallas import tpu as pltpu
from jax.experimental.pallas import tpu_sc as plsc

N_SRC0 = 10000
N_DST0 = 4000
N_DST1 = 2000
E0 = 128000
E1 = 64000
D = 128
D2 = 256

_L = 16


def _seg_max_sc(m_flat, src, dst, *, n_src, n_dst, n_fchunk, n_fct,
                chunk, rowch):
  E = src.shape[0]
  W = _L * n_fct
  n_fgroups = n_fchunk // n_fct
  fgpc = n_fgroups // 2
  assert 16 % fgpc == 0
  n_group = 16 // fgpc
  epg = E // n_group
  assert epg % chunk == 0
  n_chunks = epg // chunk
  rows_pg = n_dst // n_group
  assert rows_pg % rowch == 0 and rowch % 8 == 0

  mesh = plsc.VectorSubcoreMesh(core_axis_name="c", subcore_axis_name="s",
                                num_cores=2, num_subcores=16)

  def body(m_h, src_h, dst_h, out_h, part_h,
           srcv, dstv, idxv, rows, cmb, obuf, semg, sese, sesd, *accs):
    c = lax.axis_index("c")
    s = lax.axis_index("s")
    fg = c * fgpc + lax.rem(s, fgpc)
    g = lax.div(s, fgpc)

    col16 = lax.iota(jnp.int32, _L)
    neg_inf = jnp.full((_L,), -jnp.inf, jnp.float32)

    def init_body(r, _):
      for h in range(n_fct):
        accs[h][r, :] = neg_inf
      return 0
    lax.fori_loop(0, n_dst, init_body, 0, unroll=4)

    def start_edges(ci, p):
      eb = g * epg + ci * chunk
      pltpu.async_copy(src_h.at[pl.ds(eb, chunk)], srcv.at[p], sese.at[p])
      pltpu.async_copy(dst_h.at[pl.ds(eb, chunk)], dstv.at[p], sesd.at[p])

    def start_gather(p):
      def idx_body(j, _):
        s16 = srcv[p, pl.ds(j * _L, _L)]
        idxv[p, pl.ds(j * _L, _L)] = s16 * n_fgroups + fg
        return 0
      lax.fori_loop(0, chunk // _L, idx_body, 0, unroll=4)
      pltpu.async_copy(m_h.at[idxv.at[p]], rows.at[p], semg.at[p])

    def wait_edges(p):
      pltpu.make_async_copy(src_h.at[pl.ds(0, chunk)], srcv.at[p],
                            sese.at[p]).wait()
      pltpu.make_async_copy(dst_h.at[pl.ds(0, chunk)], dstv.at[p],
                            sesd.at[p]).wait()

    def accumulate(p):
      pltpu.make_async_copy(m_h.at[idxv.at[p]], rows.at[p], semg.at[p]).wait()

      def group_body(i, _):
        d16 = dstv[p, pl.ds(i * _L, _L)]
        for l in range(_L):
          dvec = jnp.take(d16, jnp.full((_L,), l, jnp.int32))
          for h in range(n_fct):
            cur = plsc.load_gather(accs[h], [dvec, col16])
            msg = rows[p, i * _L + l, pl.ds(h * _L, _L)]
            plsc.store_scatter(accs[h], [dvec, col16],
                               jnp.maximum(cur, msg))
        return 0
      lax.fori_loop(0, chunk // _L, group_body, 0, unroll=4)

    start_edges(0, 0)
    start_edges(1, 1)
    wait_edges(0)
    start_gather(0)

    def chunk_body(ci, _):
      p = lax.rem(ci, 2)
      pn = lax.rem(ci + 1, 2)

      @pl.when(ci + 1 < n_chunks)
      def _():
        wait_edges(pn)
        start_gather(pn)

      accumulate(p)

      @pl.when(ci + 2 < n_chunks)
      def _():
        start_edges(ci + 2, p)
      return 0
    lax.fori_loop(0, n_chunks, chunk_body, 0)

    for h in range(n_fct):
      pltpu.sync_copy(accs[h], part_h.at[g, fg * n_fct + h])
    plsc.subcore_barrier()

    r0 = g * rows_pg

    def q_body(q, _):
      rq = r0 + q * rowch
      for h in range(n_fct):
        f = fg * n_fct + h
        for gg in range(n_group):
          pltpu.sync_copy(part_h.at[gg, f, pl.ds(rq, rowch)], cmb.at[gg])

        def row_body(r, _):
          v = cmb[0, r, :]
          for gg in range(1, n_group):
            v = jnp.maximum(v, cmb[gg, r, :])
          obuf[r, :] = jnp.where(v == -jnp.inf, 0.0, v)
          return 0
        lax.fori_loop(0, rowch, row_body, 0, unroll=4)
        pltpu.sync_copy(obuf, out_h.at[f, pl.ds(rq, rowch)])
      return 0
    lax.fori_loop(0, rows_pg // rowch, q_body, 0)

  scratch = [
      pltpu.VMEM((2, chunk), jnp.int32),
      pltpu.VMEM((2, chunk), jnp.int32),
      pltpu.VMEM((2, chunk), jnp.int32),
      pltpu.VMEM((2, chunk, W), jnp.float32),
      pltpu.VMEM((n_group, rowch, _L), jnp.float32),
      pltpu.VMEM((rowch, _L), jnp.float32),
      pltpu.SemaphoreType.DMA((2,)),
      pltpu.SemaphoreType.DMA((2,)),
      pltpu.SemaphoreType.DMA((2,)),
  ] + [pltpu.VMEM((n_dst, _L), jnp.float32) for _ in range(n_fct)]
  out, _ = pl.kernel(
      body,
      out_type=(jax.ShapeDtypeStruct((n_fchunk, n_dst, _L), jnp.float32),
                jax.ShapeDtypeStruct((n_group, n_fchunk, n_dst, _L),
                                     jnp.float32)),
      mesh=mesh,
      scratch_types=scratch,
      compiler_params=pltpu.CompilerParams(
          needs_layout_passes=False, use_tc_tiling_on_sc=False),
  )(m_flat, src, dst)
  return out


def _tc_pool1(h, wt, b2d):
  def body(h_ref, w_ref, b_ref, o_ref):
    o_ref[...] = jnp.maximum(
        jnp.dot(h_ref[...], w_ref[...], preferred_element_type=jnp.float32)
        + b_ref[...], 0.0)
  return pl.pallas_call(
      body, out_shape=jax.ShapeDtypeStruct(h.shape, jnp.float32))(h, wt, b2d)


def _tc_mid(h4, nfm, ws_t, wn_fm, b1_2d, wp2_t, bp2_2d):
  def body(h_ref, n_ref, ws_ref, wn_ref, b_ref, wp_ref, bp_ref,
           h1_ref, m2_ref):
    t = jnp.dot(h_ref[...], ws_ref[...],
                preferred_element_type=jnp.float32) + b_ref[...]
    for fc in range(n_ref.shape[0]):
      t = t + jnp.dot(n_ref[fc], wn_ref[fc],
                      preferred_element_type=jnp.float32)
    h1 = jnp.maximum(t, 0.0)
    h1_ref[...] = h1
    m2_ref[...] = jnp.maximum(
        jnp.dot(h1, wp_ref[...], preferred_element_type=jnp.float32)
        + bp_ref[...], 0.0)
  return pl.pallas_call(
      body,
      out_shape=(jax.ShapeDtypeStruct((N_DST0, D2), jnp.float32),
                 jax.ShapeDtypeStruct((N_DST0, D2), jnp.float32)),
  )(h4, nfm, ws_t, wn_fm, b1_2d, wp2_t, bp2_2d)


def _tc_out(hdst, h1s, nfm, ws_t, wn_fm, b2_2d):
  def body(hd_ref, h_ref, n_ref, ws_ref, wn_ref, b_ref, o_ref):
    t = (jnp.dot(h_ref[...], ws_ref[...],
                 preferred_element_type=jnp.float32)
         + b_ref[...] + hd_ref[...])
    for fc in range(n_ref.shape[0]):
      t = t + jnp.dot(n_ref[fc][:N_DST1], wn_ref[fc],
                      preferred_element_type=jnp.float32)
    o_ref[...] = t
  return pl.pallas_call(
      body, out_shape=jax.ShapeDtypeStruct((N_DST1, D), jnp.float32),
  )(hdst, h1s, nfm, ws_t, wn_fm, b2_2d)


def kernel(h_item, h_item_dst, src0, dst0, src1, dst1,
           Wp1, bp1, Wn1, Ws1, b1, Wp2, bp2, Wn2, Ws2, b2):
  m1 = _tc_pool1(h_item, Wp1.T, bp1.reshape(1, D))
  neigh1_fm = _seg_max_sc(m1.reshape(N_SRC0 * (D // _L), _L), src0, dst0,
                          n_src=N_SRC0, n_dst=N_DST0,
                          n_fchunk=D // _L, n_fct=1, chunk=256, rowch=200)
  h1, m2 = _tc_mid(h_item[:N_DST0], neigh1_fm, Ws1.T,
                   Wn1.T.reshape(D // _L, _L, D2), b1.reshape(1, D2),
                   Wp2.T, bp2.reshape(1, D2))
  neigh2_fm = _seg_max_sc(m2.reshape(N_DST0 * (D2 // (2 * _L)), 2 * _L),
                          src1, dst1,
                          n_src=N_DST0, n_dst=2048,
                          n_fchunk=D2 // _L, n_fct=2, chunk=320, rowch=128)
  return _tc_out(h_item_dst, h1[:N_DST1], neigh2_fm, Ws2.T,
                 Wn2.T.reshape(D2 // _L, _L, D), b2.reshape(1, D))


# speedup vs baseline: 3.3546x; 1.0674x over previous
import functools

import jax
import jax.numpy as jnp
from jax import lax
from jax.experimental import pallas as pl
from jax.experimental.pallas import tpu as pltpu
from jax.experimental.pallas import tpu_sc as plsc

N_SRC0 = 10000
N_DST0 = 4000
N_DST1 = 2000
E0 = 128000
E1 = 64000
D = 128
D2 = 256

_L = 16


def _seg_max_sc(m_flat, src, dst, *, n_src, n_dst, n_fchunk, n_fct,
                chunk, rowch):
  E = src.shape[0]
  W = _L * n_fct
  n_fgroups = n_fchunk // n_fct
  fgpc = n_fgroups // 2
  assert 16 % fgpc == 0
  n_group = 16 // fgpc
  epg = E // n_group
  assert epg % chunk == 0
  n_chunks = epg // chunk
  rows_pg = n_dst // n_group
  assert rows_pg % rowch == 0 and rowch % 8 == 0

  mesh = plsc.VectorSubcoreMesh(core_axis_name="c", subcore_axis_name="s",
                                num_cores=2, num_subcores=16)

  def body(m_h, src_h, dst_h, out_h, part_h,
           srcv, dstv, idxv, rows, cmb, obuf, semg, sese, sesd, *accs):
    c = lax.axis_index("c")
    s = lax.axis_index("s")
    fg = c * fgpc + lax.rem(s, fgpc)
    g = lax.div(s, fgpc)

    col16 = lax.iota(jnp.int32, _L)
    neg_inf = jnp.full((_L,), -jnp.inf, jnp.float32)

    def init_body(r, _):
      for h in range(n_fct):
        accs[h][r, :] = neg_inf
      return 0
    lax.fori_loop(0, n_dst, init_body, 0, unroll=4)

    def start_edges(ci, p):
      eb = g * epg + ci * chunk
      pltpu.async_copy(src_h.at[pl.ds(eb, chunk)], srcv.at[p], sese.at[p])
      pltpu.async_copy(dst_h.at[pl.ds(eb, chunk)], dstv.at[p], sesd.at[p])

    def start_gather(p):
      def idx_body(j, _):
        s16 = srcv[p, pl.ds(j * _L, _L)]
        idxv[p, pl.ds(j * _L, _L)] = s16 * n_fgroups + fg
        return 0
      lax.fori_loop(0, chunk // _L, idx_body, 0, unroll=4)
      pltpu.async_copy(m_h.at[idxv.at[p]], rows.at[p], semg.at[p])

    def wait_edges(p):
      pltpu.make_async_copy(src_h.at[pl.ds(0, chunk)], srcv.at[p],
                            sese.at[p]).wait()
      pltpu.make_async_copy(dst_h.at[pl.ds(0, chunk)], dstv.at[p],
                            sesd.at[p]).wait()

    def accumulate(p):
      pltpu.make_async_copy(m_h.at[idxv.at[p]], rows.at[p], semg.at[p]).wait()

      def group_body(i, _):
        d16 = dstv[p, pl.ds(i * _L, _L)]
        for l in range(_L):
          dvec = jnp.take(d16, jnp.full((_L,), l, jnp.int32))
          for h in range(n_fct):
            cur = plsc.load_gather(accs[h], [dvec, col16])
            msg = rows[p, i * _L + l, pl.ds(h * _L, _L)]
            plsc.store_scatter(accs[h], [dvec, col16],
                               jnp.maximum(cur, msg))
        return 0
      lax.fori_loop(0, chunk // _L, group_body, 0, unroll=8)

    start_edges(0, 0)
    start_edges(1, 1)
    wait_edges(0)
    start_gather(0)

    def chunk_body(ci, _):
      p = lax.rem(ci, 2)
      pn = lax.rem(ci + 1, 2)

      @pl.when(ci + 1 < n_chunks)
      def _():
        wait_edges(pn)
        start_gather(pn)

      accumulate(p)

      @pl.when(ci + 2 < n_chunks)
      def _():
        start_edges(ci + 2, p)
      return 0
    lax.fori_loop(0, n_chunks, chunk_body, 0)

    for h in range(n_fct):
      pltpu.sync_copy(accs[h], part_h.at[g, fg * n_fct + h])
    plsc.subcore_barrier()

    r0 = g * rows_pg

    def q_body(q, _):
      rq = r0 + q * rowch
      for h in range(n_fct):
        f = fg * n_fct + h
        for gg in range(n_group):
          pltpu.sync_copy(part_h.at[gg, f, pl.ds(rq, rowch)], cmb.at[gg])

        def row_body(r, _):
          v = cmb[0, r, :]
          for gg in range(1, n_group):
            v = jnp.maximum(v, cmb[gg, r, :])
          obuf[r, :] = jnp.where(v == -jnp.inf, 0.0, v)
          return 0
        lax.fori_loop(0, rowch, row_body, 0, unroll=4)
        pltpu.sync_copy(obuf, out_h.at[f, pl.ds(rq, rowch)])
      return 0
    lax.fori_loop(0, rows_pg // rowch, q_body, 0)

  scratch = [
      pltpu.VMEM((2, chunk), jnp.int32),
      pltpu.VMEM((2, chunk), jnp.int32),
      pltpu.VMEM((2, chunk), jnp.int32),
      pltpu.VMEM((2, chunk, W), jnp.float32),
      pltpu.VMEM((n_group, rowch, _L), jnp.float32),
      pltpu.VMEM((rowch, _L), jnp.float32),
      pltpu.SemaphoreType.DMA((2,)),
      pltpu.SemaphoreType.DMA((2,)),
      pltpu.SemaphoreType.DMA((2,)),
  ] + [pltpu.VMEM((n_dst, _L), jnp.float32) for _ in range(n_fct)]
  out, _ = pl.kernel(
      body,
      out_type=(jax.ShapeDtypeStruct((n_fchunk, n_dst, _L), jnp.float32),
                jax.ShapeDtypeStruct((n_group, n_fchunk, n_dst, _L),
                                     jnp.float32)),
      mesh=mesh,
      scratch_types=scratch,
      compiler_params=pltpu.CompilerParams(
          needs_layout_passes=False, use_tc_tiling_on_sc=False),
  )(m_flat, src, dst)
  return out


def _tc_pool1(h, wt, b2d):
  def body(h_ref, w_ref, b_ref, o_ref):
    o_ref[...] = jnp.maximum(
        jnp.dot(h_ref[...], w_ref[...], preferred_element_type=jnp.float32)
        + b_ref[...], 0.0)
  return pl.pallas_call(
      body, out_shape=jax.ShapeDtypeStruct(h.shape, jnp.float32))(h, wt, b2d)


def _tc_mid(h4, nfm, ws_t, wn_fm, b1_2d, wp2_t, bp2_2d):
  def body(h_ref, n_ref, ws_ref, wn_ref, b_ref, wp_ref, bp_ref,
           h1_ref, m2_ref):
    t = jnp.dot(h_ref[...], ws_ref[...],
                preferred_element_type=jnp.float32) + b_ref[...]
    for fc in range(n_ref.shape[0]):
      t = t + jnp.dot(n_ref[fc], wn_ref[fc],
                      preferred_element_type=jnp.float32)
    h1 = jnp.maximum(t, 0.0)
    h1_ref[...] = h1
    m2_ref[...] = jnp.maximum(
        jnp.dot(h1, wp_ref[...], preferred_element_type=jnp.float32)
        + bp_ref[...], 0.0)
  return pl.pallas_call(
      body,
      out_shape=(jax.ShapeDtypeStruct((N_DST0, D2), jnp.float32),
                 jax.ShapeDtypeStruct((N_DST0, D2), jnp.float32)),
  )(h4, nfm, ws_t, wn_fm, b1_2d, wp2_t, bp2_2d)


def _tc_out(hdst, h1s, nfm, ws_t, wn_fm, b2_2d):
  def body(hd_ref, h_ref, n_ref, ws_ref, wn_ref, b_ref, o_ref):
    t = (jnp.dot(h_ref[...], ws_ref[...],
                 preferred_element_type=jnp.float32)
         + b_ref[...] + hd_ref[...])
    for fc in range(n_ref.shape[0]):
      t = t + jnp.dot(n_ref[fc][:N_DST1], wn_ref[fc],
                      preferred_element_type=jnp.float32)
    o_ref[...] = t
  return pl.pallas_call(
      body, out_shape=jax.ShapeDtypeStruct((N_DST1, D), jnp.float32),
  )(hdst, h1s, nfm, ws_t, wn_fm, b2_2d)


def kernel(h_item, h_item_dst, src0, dst0, src1, dst1,
           Wp1, bp1, Wn1, Ws1, b1, Wp2, bp2, Wn2, Ws2, b2):
  m1 = _tc_pool1(h_item, Wp1.T, bp1.reshape(1, D))
  neigh1_fm = _seg_max_sc(m1.reshape(N_SRC0 * (D // _L), _L), src0, dst0,
                          n_src=N_SRC0, n_dst=N_DST0,
                          n_fchunk=D // _L, n_fct=1, chunk=640, rowch=200)
  h1, m2 = _tc_mid(h_item[:N_DST0], neigh1_fm, Ws1.T,
                   Wn1.T.reshape(D // _L, _L, D2), b1.reshape(1, D2),
                   Wp2.T, bp2.reshape(1, D2))
  neigh2_fm = _seg_max_sc(m2.reshape(N_DST0 * (D2 // (2 * _L)), 2 * _L),
                          src1, dst1,
                          n_src=N_DST0, n_dst=2048,
                          n_fchunk=D2 // _L, n_fct=2, chunk=640, rowch=128)
  return _tc_out(h_item_dst, h1[:N_DST1], neigh2_fm, Ws2.T,
                 Wn2.T.reshape(D2 // _L, _L, D), b2.reshape(1, D))


# speedup vs baseline: 3.3856x; 1.0093x over previous
import functools

import jax
import jax.numpy as jnp
from jax import lax
from jax.experimental import pallas as pl
from jax.experimental.pallas import tpu as pltpu
from jax.experimental.pallas import tpu_sc as plsc

N_SRC0 = 10000
N_DST0 = 4000
N_DST1 = 2000
E0 = 128000
E1 = 64000
D = 128
D2 = 256

_L = 16


def _seg_max_sc(m_flat, src, dst, *, n_src, n_dst, n_fchunk, n_fct,
                chunk, rowch):
  E = src.shape[0]
  W = _L * n_fct
  n_fgroups = n_fchunk // n_fct
  fgpc = n_fgroups // 2
  assert 16 % fgpc == 0
  n_group = 16 // fgpc
  epg = E // n_group
  assert epg % chunk == 0
  n_chunks = epg // chunk
  rows_pg = n_dst // n_group
  assert rows_pg % rowch == 0 and rowch % 8 == 0

  mesh = plsc.VectorSubcoreMesh(core_axis_name="c", subcore_axis_name="s",
                                num_cores=2, num_subcores=16)

  def body(m_h, src_h, dst_h, out_h, part_h,
           srcv, dstv, idxv, rows, cmb, obuf, semg, sese, sesd, *accs):
    c = lax.axis_index("c")
    s = lax.axis_index("s")
    fg = c * fgpc + lax.rem(s, fgpc)
    g = lax.div(s, fgpc)

    col16 = lax.iota(jnp.int32, _L)
    neg_inf = jnp.full((_L,), -jnp.inf, jnp.float32)

    def init_body(r, _):
      for h in range(n_fct):
        accs[h][r, :] = neg_inf
      return 0
    lax.fori_loop(0, n_dst, init_body, 0, unroll=4)

    def start_edges(ci, p):
      eb = g * epg + ci * chunk
      pltpu.async_copy(src_h.at[pl.ds(eb, chunk)], srcv.at[p], sese.at[p])
      pltpu.async_copy(dst_h.at[pl.ds(eb, chunk)], dstv.at[p], sesd.at[p])

    def start_gather(p):
      def idx_body(j, _):
        s16 = srcv[p, pl.ds(j * _L, _L)]
        idxv[p, pl.ds(j * _L, _L)] = s16 * n_fgroups + fg
        return 0
      lax.fori_loop(0, chunk // _L, idx_body, 0, unroll=4)
      pltpu.async_copy(m_h.at[idxv.at[p]], rows.at[p], semg.at[p])

    def wait_edges(p):
      pltpu.make_async_copy(src_h.at[pl.ds(0, chunk)], srcv.at[p],
                            sese.at[p]).wait()
      pltpu.make_async_copy(dst_h.at[pl.ds(0, chunk)], dstv.at[p],
                            sesd.at[p]).wait()

    def accumulate(p):
      pltpu.make_async_copy(m_h.at[idxv.at[p]], rows.at[p], semg.at[p]).wait()

      def group_body(i, _):
        d16 = dstv[p, pl.ds(i * _L, _L)]
        for l in range(_L):
          dvec = jnp.take(d16, jnp.full((_L,), l, jnp.int32))
          for h in range(n_fct):
            cur = plsc.load_gather(accs[h], [dvec, col16])
            msg = rows[p, i * _L + l, pl.ds(h * _L, _L)]
            plsc.store_scatter(accs[h], [dvec, col16],
                               jnp.maximum(cur, msg))
        return 0
      lax.fori_loop(0, chunk // _L, group_body, 0, unroll=8)

    start_edges(0, 0)
    start_edges(1, 1)
    wait_edges(0)
    start_gather(0)

    def chunk_body(ci, _):
      p = lax.rem(ci, 2)
      pn = lax.rem(ci + 1, 2)

      @pl.when(ci + 1 < n_chunks)
      def _():
        wait_edges(pn)
        start_gather(pn)

      accumulate(p)

      @pl.when(ci + 2 < n_chunks)
      def _():
        start_edges(ci + 2, p)
      return 0
    lax.fori_loop(0, n_chunks, chunk_body, 0)

    for h in range(n_fct):
      pltpu.sync_copy(accs[h], part_h.at[g, fg * n_fct + h])
    plsc.subcore_barrier()

    r0 = g * rows_pg

    def q_body(q, _):
      rq = r0 + q * rowch
      for h in range(n_fct):
        f = fg * n_fct + h
        for gg in range(n_group):
          pltpu.sync_copy(part_h.at[gg, f, pl.ds(rq, rowch)], cmb.at[gg])

        def row_body(r, _):
          v = cmb[0, r, :]
          for gg in range(1, n_group):
            v = jnp.maximum(v, cmb[gg, r, :])
          obuf[r, :] = jnp.where(v == -jnp.inf, 0.0, v)
          return 0
        lax.fori_loop(0, rowch, row_body, 0, unroll=4)
        pltpu.sync_copy(obuf, out_h.at[f, pl.ds(rq, rowch)])
      return 0
    lax.fori_loop(0, rows_pg // rowch, q_body, 0)

  scratch = [
      pltpu.VMEM((2, chunk), jnp.int32),
      pltpu.VMEM((2, chunk), jnp.int32),
      pltpu.VMEM((2, chunk), jnp.int32),
      pltpu.VMEM((2, chunk, W), jnp.float32),
      pltpu.VMEM((n_group, rowch, _L), jnp.float32),
      pltpu.VMEM((rowch, _L), jnp.float32),
      pltpu.SemaphoreType.DMA((2,)),
      pltpu.SemaphoreType.DMA((2,)),
      pltpu.SemaphoreType.DMA((2,)),
  ] + [pltpu.VMEM((n_dst, _L), jnp.float32) for _ in range(n_fct)]
  out, _ = pl.kernel(
      body,
      out_type=(jax.ShapeDtypeStruct((n_fchunk, n_dst, _L), jnp.float32),
                jax.ShapeDtypeStruct((n_group, n_fchunk, n_dst, _L),
                                     jnp.float32)),
      mesh=mesh,
      scratch_types=scratch,
      compiler_params=pltpu.CompilerParams(
          needs_layout_passes=False, use_tc_tiling_on_sc=False),
  )(m_flat, src, dst)
  return out


def _tc_pool1(h, wt, b2d):
  def body(h_ref, w_ref, b_ref, o_ref):
    o_ref[...] = jnp.maximum(
        jnp.dot(h_ref[...], w_ref[...], preferred_element_type=jnp.float32)
        + b_ref[...], 0.0)
  return pl.pallas_call(
      body, out_shape=jax.ShapeDtypeStruct(h.shape, jnp.float32))(h, wt, b2d)


def _tc_mid(h4, nfm, ws_t, wn_fm, b1_2d, wp2_t, bp2_2d):
  def body(h_ref, n_ref, ws_ref, wn_ref, b_ref, wp_ref, bp_ref,
           h1_ref, m2_ref):
    t = jnp.dot(h_ref[...], ws_ref[...],
                preferred_element_type=jnp.float32) + b_ref[...]
    for fc in range(n_ref.shape[0]):
      t = t + jnp.dot(n_ref[fc], wn_ref[fc],
                      preferred_element_type=jnp.float32)
    h1 = jnp.maximum(t, 0.0)
    h1_ref[...] = h1
    m2_ref[...] = jnp.maximum(
        jnp.dot(h1, wp_ref[...], preferred_element_type=jnp.float32)
        + bp_ref[...], 0.0)
  return pl.pallas_call(
      body,
      out_shape=(jax.ShapeDtypeStruct((N_DST0, D2), jnp.float32),
                 jax.ShapeDtypeStruct((N_DST0, D2), jnp.float32)),
  )(h4, nfm, ws_t, wn_fm, b1_2d, wp2_t, bp2_2d)


def _tc_out(hdst, h1s, nfm, ws_t, wn_fm, b2_2d):
  def body(hd_ref, h_ref, n_ref, ws_ref, wn_ref, b_ref, o_ref):
    t = (jnp.dot(h_ref[...], ws_ref[...],
                 preferred_element_type=jnp.float32)
         + b_ref[...] + hd_ref[...])
    for fc in range(n_ref.shape[0]):
      t = t + jnp.dot(n_ref[fc][:N_DST1], wn_ref[fc],
                      preferred_element_type=jnp.float32)
    o_ref[...] = t
  return pl.pallas_call(
      body, out_shape=jax.ShapeDtypeStruct((N_DST1, D), jnp.float32),
  )(hdst, h1s, nfm, ws_t, wn_fm, b2_2d)


def kernel(h_item, h_item_dst, src0, dst0, src1, dst1,
           Wp1, bp1, Wn1, Ws1, b1, Wp2, bp2, Wn2, Ws2, b2):
  m1 = _tc_pool1(h_item, Wp1.T, bp1.reshape(1, D))
  neigh1_fm = _seg_max_sc(m1.reshape(N_SRC0 * (D // _L), _L), src0, dst0,
                          n_src=N_SRC0, n_dst=N_DST0,
                          n_fchunk=D // _L, n_fct=1, chunk=800, rowch=200)
  h1, m2 = _tc_mid(h_item[:N_DST0], neigh1_fm, Ws1.T,
                   Wn1.T.reshape(D // _L, _L, D2), b1.reshape(1, D2),
                   Wp2.T, bp2.reshape(1, D2))
  neigh2_fm = _seg_max_sc(m2.reshape(N_DST0 * (D2 // (2 * _L)), 2 * _L),
                          src1, dst1,
                          n_src=N_DST0, n_dst=2048,
                          n_fchunk=D2 // _L, n_fct=2, chunk=640, rowch=128)
  return _tc_out(h_item_dst, h1[:N_DST1], neigh2_fm, Ws2.T,
                 Wn2.T.reshape(D2 // _L, _L, D), b2.reshape(1, D))
